# baseline (speedup 1.0000x reference)
import jax
import jax.numpy as jnp
from jax.experimental import pallas as pl
from jax.experimental.pallas import tpu as pltpu

_NSEM = 8


def _dma_body(z_ref, o1_ref, o2_ref, *sems):
    n = z_ref.shape[0]
    ch = z_ref.shape[1] // 2
    copies = []
    for i in range(n):
        copies.append(pltpu.make_async_copy(
            z_ref.at[i, pl.ds(0, ch)], o1_ref.at[i], sems[(2 * i) % _NSEM]))
        copies.append(pltpu.make_async_copy(
            z_ref.at[i, pl.ds(ch, ch)], o2_ref.at[i], sems[(2 * i + 1) % _NSEM]))
    for cp in copies:
        cp.start()
    for cp in copies:
        cp.wait()


def kernel(z):
    n, c, h, w = z.shape
    ch = c // 2

    z1, z2 = pl.pallas_call(
        _dma_body,
        in_specs=[pl.BlockSpec(memory_space=pl.ANY)],
        out_specs=[
            pl.BlockSpec(memory_space=pl.ANY),
            pl.BlockSpec(memory_space=pl.ANY),
        ],
        out_shape=[
            jax.ShapeDtypeStruct((n, ch, h, w), z.dtype),
            jax.ShapeDtypeStruct((n, ch, h, w), z.dtype),
        ],
        scratch_shapes=[pltpu.SemaphoreType.DMA] * _NSEM,
    )(z)

    log_det = jnp.zeros((), z.dtype)
    return (z1, z2, log_det)


# speedup vs baseline: 14.4001x; 14.4001x over previous
import jax
import jax.numpy as jnp
from jax.experimental import pallas as pl
from jax.experimental.pallas import tpu as pltpu

_NBUF = 8


def _ring_body(z_ref, o1_ref, o2_ref, *scratch):
    bufs = scratch[:_NBUF]
    isems = scratch[_NBUF:2 * _NBUF]
    osems = scratch[2 * _NBUF:3 * _NBUF]
    n = z_ref.shape[0]
    ch = z_ref.shape[1] // 2
    units = [(i, half) for i in range(n) for half in (0, 1)]
    nu = len(units)

    def in_copy(u):
        i, half = units[u]
        src = z_ref.at[pl.ds(i, 1), pl.ds(half * ch, ch)]
        return pltpu.make_async_copy(src, bufs[u % _NBUF], isems[u % _NBUF])

    def out_copy(u):
        i, half = units[u]
        dst_ref = o1_ref if half == 0 else o2_ref
        dst = dst_ref.at[pl.ds(i, 1)]
        return pltpu.make_async_copy(bufs[u % _NBUF], dst, osems[u % _NBUF])

    for u in range(min(_NBUF, nu)):
        in_copy(u).start()
    for u in range(nu):
        in_copy(u).wait()
        out_copy(u).start()
        if u + _NBUF < nu:
            out_copy(u).wait()
            in_copy(u + _NBUF).start()
    for u in range(max(nu - _NBUF, 0), nu):
        out_copy(u).wait()


def kernel(z):
    n, c, h, w = z.shape
    ch = c // 2

    z1, z2 = pl.pallas_call(
        _ring_body,
        in_specs=[pl.BlockSpec(memory_space=pl.ANY)],
        out_specs=[
            pl.BlockSpec(memory_space=pl.ANY),
            pl.BlockSpec(memory_space=pl.ANY),
        ],
        out_shape=[
            jax.ShapeDtypeStruct((n, ch, h, w), z.dtype),
            jax.ShapeDtypeStruct((n, ch, h, w), z.dtype),
        ],
        scratch_shapes=(
            [pltpu.VMEM((1, ch, h, w), jnp.float32)] * _NBUF
            + [pltpu.SemaphoreType.DMA] * (2 * _NBUF)
        ),
    )(z)

    log_det = jnp.zeros((), z.dtype)
    return (z1, z2, log_det)


# speedup vs baseline: 77.2407x; 5.3639x over previous
import jax
import jax.numpy as jnp
from jax.experimental import pallas as pl


def _split_body(zt_ref, a_ref, b_ref):
    ch = a_ref.shape[-1]
    a_ref[...] = zt_ref[:, :, :, :ch]
    b_ref[...] = zt_ref[:, :, :, ch:]


def kernel(z):
    n, c, h, w = z.shape
    ch = c // 2
    zt = jnp.transpose(z, (0, 2, 3, 1))

    o1, o2 = pl.pallas_call(
        _split_body,
        grid=(n,),
        in_specs=[pl.BlockSpec((1, h, w, c), lambda i: (i, 0, 0, 0))],
        out_specs=[
            pl.BlockSpec((1, h, w, ch), lambda i: (i, 0, 0, 0)),
            pl.BlockSpec((1, h, w, ch), lambda i: (i, 0, 0, 0)),
        ],
        out_shape=[
            jax.ShapeDtypeStruct((n, h, w, ch), z.dtype),
            jax.ShapeDtypeStruct((n, h, w, ch), z.dtype),
        ],
    )(zt)

    z1 = jnp.transpose(o1, (0, 3, 1, 2))
    z2 = jnp.transpose(o2, (0, 3, 1, 2))
    log_det = jnp.zeros((), z.dtype)
    return (z1, z2, log_det)
